# baseline (speedup 1.0000x reference)
import functools

import jax
import jax.numpy as jnp
from jax import lax
from jax.experimental import pallas as pl
from jax.experimental.pallas import tpu as pltpu
from jax.experimental.pallas import tpu_sc as plsc

_NW = 32
_CH = 80
_TB = 5120
_TP = 5
_NSLICE = 2
_L = 16


def _sc_gather_bf16(word_emb, idx):
  nch = idx.shape[1]
  d = word_emb.shape[1]
  n2 = _NW * nch * (_CH // 2)
  mesh = plsc.VectorSubcoreMesh(core_axis_name="c", subcore_axis_name="s")

  @functools.partial(
      pl.kernel, mesh=mesh,
      compiler_params=pltpu.CompilerParams(use_tc_tiling_on_sc=True,
                                           needs_layout_passes=False),
      out_type=jax.ShapeDtypeStruct((n2, d), jnp.int32),
      scratch_types=[
          pltpu.VMEM((nch, _CH), jnp.int32),
          pltpu.VMEM((_CH, d), jnp.float32),
          pltpu.VMEM((_CH, d), jnp.float32),
          pltpu.VMEM((_CH // 2, d), jnp.int32),
          pltpu.VMEM((_CH // 2, d), jnp.int32),
          pltpu.SemaphoreType.DMA,
          pltpu.SemaphoreType.DMA,
          pltpu.SemaphoreType.DMA,
          pltpu.SemaphoreType.DMA,
      ],
  )
  def k(table_hbm, idx_hbm, out_hbm, idx_v, rows0, rows1, pk0, pk1,
        g0, g1, w0, w1):
    wid = lax.axis_index("s") * 2 + lax.axis_index("c")
    pltpu.sync_copy(idx_hbm.at[wid], idx_v)
    base = wid * (nch * (_CH // 2))
    rows = (rows0, rows1)
    pk = (pk0, pk1)
    gsem = (g0, g1)
    wsem = (w0, w1)

    def fire_gather(c):
      return pltpu.async_copy(table_hbm.at[idx_v.at[c]], rows[c % 2],
                              gsem[c % 2])

    def fire_write(c):
      return pltpu.async_copy(pk[c % 2],
                              out_hbm.at[pl.ds(base + c * (_CH // 2),
                                               _CH // 2)],
                              wsem[c % 2])

    def pack_chunk(c):
      src = rows[c % 2]
      dst = pk[c % 2]

      def pair(p, carry):
        for half in range(2):
          r = 2 * p + half
          for g in range(d // 32):
            a = src[r, pl.ds(g * 32, _L)]
            b = src[r, pl.ds(g * 32 + _L, _L)]
            w = plsc.bitcast(
                plsc.pack(a, b, format=plsc.PackFormat.INTERLEAVED),
                jnp.int32)
            dst[p, pl.ds(half * (d // 2) + g * _L, _L)] = w
        return carry

      lax.fori_loop(0, _CH // 2, pair, 0)

    g = fire_gather(0)
    w_prev = [None, None]
    for c in range(nch):
      if c + 1 < nch:
        if w_prev[(c + 1) % 2] is not None:
          w_prev[(c + 1) % 2].wait()
        g_next = fire_gather(c + 1)
      g.wait()
      pack_chunk(c)
      w_prev[c % 2] = fire_write(c)
      if c + 1 < nch:
        g = g_next
    w_prev[(nch - 2) % 2].wait()
    w_prev[(nch - 1) % 2].wait()

  return k(word_emb, idx)


def _tc_compute(word_ref, vc_ref, jc_ref, tbl_ref, pos_ref, type_ref, g_ref,
                b_ref, m_ref, o_ref):
  x = word_ref[...].astype(jnp.float32)
  r = tbl_ref.shape[0]
  tb, d = x.shape
  bsz = tb // _TP
  iota = lax.broadcasted_iota(jnp.int32, (r, tb), 0)
  oh = ((vc_ref[0] == iota) | (jc_ref[0] == iota)).astype(jnp.float32)
  add = lax.dot_general(oh, tbl_ref[...], (((0,), (0,)), ((), ())),
                        preferred_element_type=jnp.float32)
  x = (x + add).reshape(_TP, bsz, d)
  x = x + (pos_ref[...] + type_ref[0:1, :][None])
  mean = jnp.mean(x, axis=2, keepdims=True)
  xc = x - mean
  var = jnp.mean(xc * xc, axis=2, keepdims=True)
  y = xc * lax.rsqrt(var + 1e-12)
  y = (y * g_ref[...][None] + b_ref[...][None]).reshape(tb, d)
  o_ref[...] = jnp.dot(y, m_ref[...], preferred_element_type=jnp.float32)


def _tc_body_first(word_ref, vc_ref, jc_ref, tbl_ref, pos_ref, type_ref,
                   g_ref, b_ref, m_ref, o_ref):
  _tc_compute(word_ref, vc_ref, jc_ref, tbl_ref, pos_ref, type_ref, g_ref,
              b_ref, m_ref, o_ref)


def _tc_body_acc(acc_ref, word_ref, vc_ref, jc_ref, tbl_ref, pos_ref,
                 type_ref, g_ref, b_ref, m_ref, o_ref):
  del acc_ref
  _tc_compute(word_ref, vc_ref, jc_ref, tbl_ref, pos_ref, type_ref, g_ref,
              b_ref, m_ref, o_ref)


def _tc_post(word_bf, vc, jc, table, pos3, type_emb, gamma, beta, unperm,
             out_prev, off, n_total):
  ns, d = word_bf.shape
  nb_s = ns // _TB
  r = table.shape[0]
  in_specs = [
      pl.BlockSpec((_TB, d), lambda i: (i, 0)),
      pl.BlockSpec((1, 1, _TB), lambda i: (i, 0, 0)),
      pl.BlockSpec((1, 1, _TB), lambda i: (i, 0, 0)),
      pl.BlockSpec((r, d), lambda i: (0, 0)),
      pl.BlockSpec((_TP, 1, d), lambda i, off=off: (i + off, 0, 0)),
      pl.BlockSpec(type_emb.shape, lambda i: (0, 0)),
      pl.BlockSpec((1, d), lambda i: (0, 0)),
      pl.BlockSpec((1, d), lambda i: (0, 0)),
      pl.BlockSpec((d, d), lambda i: (0, 0)),
  ]
  args = [word_bf, vc, jc, table, pos3, type_emb, gamma, beta, unperm]
  kwargs = {}
  if out_prev is None:
    body = _tc_body_first
  else:
    body = _tc_body_acc
    in_specs = [pl.BlockSpec(memory_space=pl.ANY)] + in_specs
    args = [out_prev] + args
    kwargs["input_output_aliases"] = {0: 0}
  return pl.pallas_call(
      body,
      grid=(nb_s,),
      in_specs=in_specs,
      out_specs=pl.BlockSpec((_TB, d), lambda i, off=off: (i + off, 0)),
      out_shape=jax.ShapeDtypeStruct((n_total, d), jnp.float32),
      **kwargs,
  )(*args)


def kernel(input_ids, v_gene_ids, j_gene_ids, word_emb, pos_emb, type_emb,
           v_emb, j_emb, ln_gamma, ln_beta):
  b, l = input_ids.shape
  d = word_emb.shape[1]
  n = b * l
  nb = n // _TB
  nv = v_emb.shape[0]

  m_col = jnp.arange(d, dtype=jnp.int32)
  grp, rem = m_col // 32, m_col % 32
  o = grp * 32 + rem // 2 + _L * (rem % 2)
  unperm = (o[:, None] == m_col[None, :]).astype(jnp.float32)

  flat = input_ids.T.astype(jnp.int32).reshape(n)
  vc = v_gene_ids.T.astype(jnp.int32).reshape(nb, 1, _TB)
  jc = (j_gene_ids.T.astype(jnp.int32) + nv).reshape(nb, 1, _TB)
  table = jnp.concatenate([v_emb, j_emb], axis=0)[:, o]
  pos3 = pos_emb[:l][:, o].reshape(l, 1, d)
  type_p = type_emb[:, o]
  gamma = ln_gamma[o].reshape(1, d)
  beta = ln_beta[o].reshape(1, d)

  ns = n // _NSLICE
  nb_s = nb // _NSLICE
  out = None
  for s in range(_NSLICE):
    idx_s = lax.slice_in_dim(flat, s * ns, (s + 1) * ns).reshape(
        _NW, ns // (_NW * _CH), _CH)
    wi_s = _sc_gather_bf16(word_emb, idx_s)
    wb_s = lax.bitcast_convert_type(wi_s, jnp.bfloat16).reshape(ns, d)
    out = _tc_post(wb_s, vc[s * nb_s:(s + 1) * nb_s],
                   jc[s * nb_s:(s + 1) * nb_s], table, pos3, type_p,
                   gamma, beta, unperm, out, s * nb_s, n)
  return out.reshape(l, b, d).transpose(1, 0, 2)


# speedup vs baseline: 35.6056x; 35.6056x over previous
import functools

import jax
import jax.numpy as jnp
from jax import lax
from jax.experimental import pallas as pl
from jax.experimental.pallas import tpu as pltpu
from jax.experimental.pallas import tpu_sc as plsc

_NW = 32
_CH = 80
_TB = 5120
_TP = 5
_NSLICE = 2
_NBUF = 4


def _sc_gather(word_emb, idx):
  nch = idx.shape[1]
  n = _NW * nch * _CH
  d = word_emb.shape[1]
  mesh = plsc.VectorSubcoreMesh(core_axis_name="c", subcore_axis_name="s")

  @functools.partial(
      pl.kernel, mesh=mesh,
      compiler_params=pltpu.CompilerParams(use_tc_tiling_on_sc=True),
      out_type=jax.ShapeDtypeStruct((n, d), jnp.float32),
      scratch_types=[
          pltpu.VMEM((nch, _CH), jnp.int32),
      ] + [pltpu.VMEM((_CH, d), jnp.float32) for _ in range(_NBUF)]
        + [pltpu.SemaphoreType.DMA for _ in range(2 * _NBUF)],
  )
  def k(table_hbm, idx_hbm, out_hbm, idx_v, *bufs_and_sems):
    rows = bufs_and_sems[:_NBUF]
    gsem = bufs_and_sems[_NBUF:2 * _NBUF]
    wsem = bufs_and_sems[2 * _NBUF:]
    wid = lax.axis_index("s") * 2 + lax.axis_index("c")
    pltpu.sync_copy(idx_hbm.at[wid], idx_v)
    base = wid * (nch * _CH)

    def fire_gather(c):
      return pltpu.async_copy(table_hbm.at[idx_v.at[c]], rows[c % _NBUF],
                              gsem[c % _NBUF])

    def fire_write(c):
      return pltpu.async_copy(rows[c % _NBUF],
                              out_hbm.at[pl.ds(base + c * _CH, _CH)],
                              wsem[c % _NBUF])

    gs = {}
    ws = {}
    for c in range(min(_NBUF - 1, nch)):
      gs[c] = fire_gather(c)
    for c in range(nch):
      nxt = c + _NBUF - 1
      if nxt < nch:
        if c - 1 >= 0:
          ws[c - 1].wait()
        gs[nxt] = fire_gather(nxt)
      gs[c].wait()
      ws[c] = fire_write(c)
    for c in range(max(0, nch - _NBUF), nch):
      ws[c].wait()

  return k(word_emb, idx)


def _tc_compute(word_ref, vc_ref, jc_ref, tbl_ref, pos_ref, type_ref, g_ref,
                b_ref, o_ref):
  x = word_ref[...]
  r = tbl_ref.shape[0]
  tb, d = x.shape
  bsz = tb // _TP
  iota = lax.broadcasted_iota(jnp.int32, (r, tb), 0)
  oh = ((vc_ref[0] == iota) | (jc_ref[0] == iota)).astype(jnp.float32)
  add = lax.dot_general(oh, tbl_ref[...], (((0,), (0,)), ((), ())),
                        preferred_element_type=jnp.float32)
  x = (x + add).reshape(_TP, bsz, d)
  x = x + (pos_ref[...] + type_ref[0:1, :][None])
  mean = jnp.mean(x, axis=2, keepdims=True)
  xc = x - mean
  var = jnp.mean(xc * xc, axis=2, keepdims=True)
  y = xc * lax.rsqrt(var + 1e-12)
  o_ref[...] = (y * g_ref[...][None] + b_ref[...][None]).reshape(tb, d)


def _tc_body_first(word_ref, vc_ref, jc_ref, tbl_ref, pos_ref, type_ref,
                   g_ref, b_ref, o_ref):
  _tc_compute(word_ref, vc_ref, jc_ref, tbl_ref, pos_ref, type_ref, g_ref,
              b_ref, o_ref)


def _tc_body_acc(acc_ref, word_ref, vc_ref, jc_ref, tbl_ref, pos_ref,
                 type_ref, g_ref, b_ref, o_ref):
  del acc_ref
  _tc_compute(word_ref, vc_ref, jc_ref, tbl_ref, pos_ref, type_ref, g_ref,
              b_ref, o_ref)


def _tc_post(word_rows, vc, jc, table, pos3, type_emb, gamma, beta,
             out_prev, off, n_total):
  ns, d = word_rows.shape
  nb_s = ns // _TB
  r = table.shape[0]
  in_specs = [
      pl.BlockSpec((_TB, d), lambda i: (i, 0)),
      pl.BlockSpec((1, 1, _TB), lambda i: (i, 0, 0)),
      pl.BlockSpec((1, 1, _TB), lambda i: (i, 0, 0)),
      pl.BlockSpec((r, d), lambda i: (0, 0)),
      pl.BlockSpec((_TP, 1, d), lambda i, off=off: (i + off, 0, 0)),
      pl.BlockSpec(type_emb.shape, lambda i: (0, 0)),
      pl.BlockSpec((1, d), lambda i: (0, 0)),
      pl.BlockSpec((1, d), lambda i: (0, 0)),
  ]
  args = [word_rows, vc, jc, table, pos3, type_emb, gamma, beta]
  kwargs = {}
  if out_prev is None:
    body = _tc_body_first
  else:
    body = _tc_body_acc
    in_specs = [pl.BlockSpec(memory_space=pl.ANY)] + in_specs
    args = [out_prev] + args
    kwargs["input_output_aliases"] = {0: 0}
  return pl.pallas_call(
      body,
      grid=(nb_s,),
      in_specs=in_specs,
      out_specs=pl.BlockSpec((_TB, d), lambda i, off=off: (i + off, 0)),
      out_shape=jax.ShapeDtypeStruct((n_total, d), jnp.float32),
      **kwargs,
  )(*args)


def kernel(input_ids, v_gene_ids, j_gene_ids, word_emb, pos_emb, type_emb,
           v_emb, j_emb, ln_gamma, ln_beta):
  b, l = input_ids.shape
  d = word_emb.shape[1]
  n = b * l
  nb = n // _TB
  nv = v_emb.shape[0]

  flat = input_ids.T.astype(jnp.int32).reshape(n)
  vc = v_gene_ids.T.astype(jnp.int32).reshape(nb, 1, _TB)
  jc = (j_gene_ids.T.astype(jnp.int32) + nv).reshape(nb, 1, _TB)
  table = jnp.concatenate([v_emb, j_emb], axis=0)
  pos3 = pos_emb[:l].reshape(l, 1, d)
  gamma = ln_gamma.reshape(1, d)
  beta = ln_beta.reshape(1, d)

  ns = n // _NSLICE
  nb_s = nb // _NSLICE
  out = None
  for s in range(_NSLICE):
    idx_s = lax.slice_in_dim(flat, s * ns, (s + 1) * ns).reshape(
        _NW, ns // (_NW * _CH), _CH)
    wr_s = _sc_gather(word_emb, idx_s)
    out = _tc_post(wr_s, vc[s * nb_s:(s + 1) * nb_s],
                   jc[s * nb_s:(s + 1) * nb_s], table, pos3, type_emb,
                   gamma, beta, out, s * nb_s, n)
  return out.reshape(l, b, d).transpose(1, 0, 2)
